# Initial kernel scaffold; baseline (speedup 1.0000x reference)
import jax
import jax.numpy as jnp
from jax.experimental import pallas as pl


def kernel(x, t, time_embed, W, b):
    raise NotImplementedError("write your pallas kernel here")


# speedup vs baseline: 1.0826x; 1.0826x over previous
import functools

import jax
import jax.numpy as jnp
from jax import lax
from jax.experimental import pallas as pl
from jax.experimental.pallas import tpu as pltpu
from jax.experimental.pallas import tpu_sc as plsc

DIM = 64
NUM_WORKERS = 32
GATHER_CHUNK = 128


def _project_body(te_ref, w2_ref, b_ref, p_ref):
    p_ref[...] = (
        lax.dot_general(
            te_ref[...], w2_ref[...],
            (((1,), (1,)), ((), ())),
            preferred_element_type=jnp.float32,
        )
        + b_ref[...]
    )


def _project_table(time_embed, w2, b_row):
    n = time_embed.shape[0]
    return pl.pallas_call(
        _project_body,
        out_shape=jax.ShapeDtypeStruct((n, DIM), jnp.float32),
    )(time_embed, w2, b_row)


def _sc_gather(P, t):
    batch = t.shape[0]
    b_per_w = batch // NUM_WORKERS
    nchunks = b_per_w // GATHER_CHUNK
    mesh = plsc.VectorSubcoreMesh(core_axis_name="c", subcore_axis_name="s")

    @functools.partial(
        pl.kernel,
        mesh=mesh,
        compiler_params=pltpu.CompilerParams(use_tc_tiling_on_sc=False),
        out_type=jax.ShapeDtypeStruct((batch, DIM), jnp.float32),
        scratch_types=[
            pltpu.VMEM((nchunks, GATHER_CHUNK), jnp.int32),
            pltpu.VMEM((nchunks, GATHER_CHUNK, DIM), jnp.float32),
            pltpu.SemaphoreType.DMA,
        ],
    )
    def gather_kernel(p_hbm, t_hbm, out_hbm, idx_v, rows_v, sem):
        wid = lax.axis_index("s") * 2 + lax.axis_index("c")
        base = wid * b_per_w
        for j in range(nchunks):
            pltpu.sync_copy(
                t_hbm.at[pl.ds(base + j * GATHER_CHUNK, GATHER_CHUNK)],
                idx_v.at[j],
            )
        copies = [
            pltpu.async_copy(p_hbm.at[idx_v.at[j]], rows_v.at[j], sem)
            for j in range(nchunks)
        ]
        for c in copies:
            c.wait()
        for j in range(nchunks):
            pltpu.sync_copy(
                rows_v.at[j],
                out_hbm.at[pl.ds(base + j * GATHER_CHUNK, GATHER_CHUNK)],
            )

    return gather_kernel(P, t)


def _matmul_add_body(x_ref, w1_ref, g_ref, o_ref):
    o_ref[...] = (
        lax.dot_general(
            x_ref[...], w1_ref[...],
            (((1,), (1,)), ((), ())),
            preferred_element_type=jnp.float32,
        )
        + g_ref[...]
    )


def _matmul_add(x, w1, g, block_rows=2048):
    batch = x.shape[0]
    grid = batch // block_rows
    return pl.pallas_call(
        _matmul_add_body,
        grid=(grid,),
        in_specs=[
            pl.BlockSpec((block_rows, DIM), lambda i: (i, 0)),
            pl.BlockSpec((DIM, DIM), lambda i: (0, 0)),
            pl.BlockSpec((block_rows, DIM), lambda i: (i, 0)),
        ],
        out_specs=pl.BlockSpec((block_rows, DIM), lambda i: (i, 0)),
        out_shape=jax.ShapeDtypeStruct((batch, DIM), jnp.float32),
    )(x, w1, g)


def kernel(x, t, time_embed, W, b):
    w1 = W[:, :DIM]
    w2 = W[:, DIM:]
    b_row = b.reshape(1, DIM)
    P = _project_table(time_embed, w2, b_row)
    G = _sc_gather(P, t.astype(jnp.int32))
    return _matmul_add(x, w1, G)


# speedup vs baseline: 1.2089x; 1.1166x over previous
import functools

import jax
import jax.numpy as jnp
from jax import lax
from jax.experimental import pallas as pl
from jax.experimental.pallas import tpu as pltpu
from jax.experimental.pallas import tpu_sc as plsc

DIM = 64
NUM_WORKERS = 32
GATHER_CHUNK = 128


def _sc_gather(table, t):
    batch = t.shape[0]
    b_per_w = batch // NUM_WORKERS
    nchunks = b_per_w // GATHER_CHUNK
    mesh = plsc.VectorSubcoreMesh(core_axis_name="c", subcore_axis_name="s")

    @functools.partial(
        pl.kernel,
        mesh=mesh,
        compiler_params=pltpu.CompilerParams(use_tc_tiling_on_sc=False),
        out_type=jax.ShapeDtypeStruct((batch, DIM), jnp.float32),
        scratch_types=[
            pltpu.VMEM((nchunks, GATHER_CHUNK), jnp.int32),
            pltpu.VMEM((nchunks, GATHER_CHUNK, DIM), jnp.float32),
            pltpu.SemaphoreType.DMA,
            pltpu.SemaphoreType.DMA,
        ],
    )
    def gather_kernel(tab_hbm, t_hbm, out_hbm, idx_v, rows_v, sem_i, sem_g):
        wid = lax.axis_index("s") * 2 + lax.axis_index("c")
        base = wid * b_per_w
        idx_copies = [
            pltpu.async_copy(
                t_hbm.at[pl.ds(base + j * GATHER_CHUNK, GATHER_CHUNK)],
                idx_v.at[j],
                sem_i,
            )
            for j in range(nchunks)
        ]
        gathers = []
        for j in range(nchunks):
            idx_copies[j].wait()
            gathers.append(
                pltpu.async_copy(tab_hbm.at[idx_v.at[j]], rows_v.at[j], sem_g)
            )
        for j in range(nchunks):
            gathers[j].wait()
            pltpu.sync_copy(
                rows_v.at[j],
                out_hbm.at[pl.ds(base + j * GATHER_CHUNK, GATHER_CHUNK)],
            )

    return gather_kernel(table, t)


def _fused_body(x_ref, g_ref, w_ref, b_ref, o_ref):
    xw = lax.dot_general(
        x_ref[...], w_ref[:, :DIM],
        (((1,), (1,)), ((), ())),
        preferred_element_type=jnp.float32,
    )
    gw = lax.dot_general(
        g_ref[...], w_ref[:, DIM:],
        (((1,), (1,)), ((), ())),
        preferred_element_type=jnp.float32,
    )
    o_ref[...] = xw + gw + b_ref[...]


def _fused_matmul(x, g, W, b_row, block_rows=2048):
    batch = x.shape[0]
    grid = batch // block_rows
    return pl.pallas_call(
        _fused_body,
        grid=(grid,),
        in_specs=[
            pl.BlockSpec((block_rows, DIM), lambda i: (i, 0)),
            pl.BlockSpec((block_rows, DIM), lambda i: (i, 0)),
            pl.BlockSpec((DIM, 2 * DIM), lambda i: (0, 0)),
            pl.BlockSpec((1, DIM), lambda i: (0, 0)),
        ],
        out_specs=pl.BlockSpec((block_rows, DIM), lambda i: (i, 0)),
        out_shape=jax.ShapeDtypeStruct((batch, DIM), jnp.float32),
    )(x, g, W, b_row)


def kernel(x, t, time_embed, W, b):
    G = _sc_gather(time_embed, t.astype(jnp.int32))
    return _fused_matmul(x, G, W, b.reshape(1, DIM))


# speedup vs baseline: 1.2115x; 1.0022x over previous
import functools

import jax
import jax.numpy as jnp
from jax import lax
from jax.experimental import pallas as pl
from jax.experimental.pallas import tpu as pltpu
from jax.experimental.pallas import tpu_sc as plsc

DIM = 64
NUM_WORKERS = 32
GATHER_CHUNK = 128


def _sc_gather(table, t):
    batch = t.shape[0]
    b_per_w = batch // NUM_WORKERS
    nchunks = b_per_w // GATHER_CHUNK
    mesh = plsc.VectorSubcoreMesh(core_axis_name="c", subcore_axis_name="s")

    @functools.partial(
        pl.kernel,
        mesh=mesh,
        compiler_params=pltpu.CompilerParams(use_tc_tiling_on_sc=False),
        out_type=jax.ShapeDtypeStruct((batch, DIM), jnp.float32),
        scratch_types=[
            pltpu.VMEM((b_per_w,), jnp.int32),
            pltpu.VMEM((b_per_w, DIM), jnp.float32),
            pltpu.SemaphoreType.DMA,
            pltpu.SemaphoreType.DMA,
        ],
    )
    def gather_kernel(tab_hbm, t_hbm, out_hbm, idx_v, rows_v, sem_g, sem_w):
        wid = lax.axis_index("s") * 2 + lax.axis_index("c")
        base = wid * b_per_w
        half = b_per_w // 2
        pltpu.sync_copy(t_hbm.at[pl.ds(base, b_per_w)], idx_v)
        gathers = [
            pltpu.async_copy(
                tab_hbm.at[idx_v.at[pl.ds(j * GATHER_CHUNK, GATHER_CHUNK)]],
                rows_v.at[pl.ds(j * GATHER_CHUNK, GATHER_CHUNK)],
                sem_g,
            )
            for j in range(nchunks)
        ]
        for j in range(nchunks // 2):
            gathers[j].wait()
        w0 = pltpu.async_copy(
            rows_v.at[pl.ds(0, half)], out_hbm.at[pl.ds(base, half)], sem_w
        )
        for j in range(nchunks // 2, nchunks):
            gathers[j].wait()
        w1 = pltpu.async_copy(
            rows_v.at[pl.ds(half, half)],
            out_hbm.at[pl.ds(base + half, half)],
            sem_w,
        )
        w0.wait()
        w1.wait()

    return gather_kernel(table, t)


def _fused_body(x_ref, g_ref, w_ref, b_ref, o_ref):
    xw = lax.dot_general(
        x_ref[...], w_ref[:, :DIM],
        (((1,), (1,)), ((), ())),
        preferred_element_type=jnp.float32,
    )
    gw = lax.dot_general(
        g_ref[...], w_ref[:, DIM:],
        (((1,), (1,)), ((), ())),
        preferred_element_type=jnp.float32,
    )
    o_ref[...] = xw + gw + b_ref[...]


def _fused_matmul(x, g, W, b_row, block_rows=2048):
    batch = x.shape[0]
    grid = batch // block_rows
    return pl.pallas_call(
        _fused_body,
        grid=(grid,),
        in_specs=[
            pl.BlockSpec((block_rows, DIM), lambda i: (i, 0)),
            pl.BlockSpec((block_rows, DIM), lambda i: (i, 0)),
            pl.BlockSpec((DIM, 2 * DIM), lambda i: (0, 0)),
            pl.BlockSpec((1, DIM), lambda i: (0, 0)),
        ],
        out_specs=pl.BlockSpec((block_rows, DIM), lambda i: (i, 0)),
        out_shape=jax.ShapeDtypeStruct((batch, DIM), jnp.float32),
    )(x, g, W, b_row)


def kernel(x, t, time_embed, W, b):
    G = _sc_gather(time_embed, t.astype(jnp.int32))
    return _fused_matmul(x, G, W, b.reshape(1, DIM))


# speedup vs baseline: 1.9347x; 1.5969x over previous
import functools

import jax
import jax.numpy as jnp
from jax import lax
from jax.experimental import pallas as pl
from jax.experimental.pallas import tpu as pltpu
from jax.experimental.pallas import tpu_sc as plsc

DIM = 64
NUM_WORKERS = 32
GATHER_CHUNK = 128
BLOCK = 2048


def _sc_gather_packed(table, t):
    batch = t.shape[0]
    b_per_w = batch // NUM_WORKERS
    nchunks = b_per_w // GATHER_CHUNK
    mesh = plsc.VectorSubcoreMesh(core_axis_name="c", subcore_axis_name="s")

    @functools.partial(
        pl.kernel,
        mesh=mesh,
        compiler_params=pltpu.CompilerParams(use_tc_tiling_on_sc=False),
        out_type=jax.ShapeDtypeStruct((batch // 2, 2 * DIM), jnp.float32),
        scratch_types=[
            pltpu.VMEM((b_per_w,), jnp.int32),
            pltpu.VMEM((b_per_w, DIM), jnp.float32),
            pltpu.SemaphoreType.DMA,
        ],
    )
    def gather_kernel(tab_hbm, t_hbm, out_hbm, idx_v, rows_v, sem_g):
        wid = lax.axis_index("s") * 2 + lax.axis_index("c")
        base = wid * b_per_w
        rowbase = 1024 * (wid // 4) + 512 * (wid % 2)
        colbase = DIM * ((wid % 4) // 2)
        pltpu.sync_copy(t_hbm.at[pl.ds(base, b_per_w)], idx_v)
        gathers = [
            pltpu.async_copy(
                tab_hbm.at[idx_v.at[pl.ds(j * GATHER_CHUNK, GATHER_CHUNK)]],
                rows_v.at[pl.ds(j * GATHER_CHUNK, GATHER_CHUNK)],
                sem_g,
            )
            for j in range(nchunks)
        ]
        for g in gathers:
            g.wait()
        pltpu.sync_copy(
            rows_v,
            out_hbm.at[pl.ds(rowbase, b_per_w), pl.ds(colbase, DIM)],
        )

    return gather_kernel(table, t)


def _fused_body(xt_ref, g_ref, w_ref, b_ref, o_ref):
    half = BLOCK // 2
    xw = lax.dot_general(
        w_ref[:, :DIM], xt_ref[...],
        (((1,), (0,)), ((), ())),
        preferred_element_type=jnp.float32,
    )
    yev = lax.dot_general(
        w_ref[:, DIM:], g_ref[:, :DIM],
        (((1,), (1,)), ((), ())),
        preferred_element_type=jnp.float32,
    )
    yod = lax.dot_general(
        w_ref[:, DIM:], g_ref[:, DIM:],
        (((1,), (1,)), ((), ())),
        preferred_element_type=jnp.float32,
    )
    bias = b_ref[...]
    o_ref[:, :half] = xw[:, :half] + yev + bias
    o_ref[:, half:] = xw[:, half:] + yod + bias


def _fused_matmul_t(xt, g128, W, b_col):
    batch = xt.shape[1]
    grid = batch // BLOCK
    return pl.pallas_call(
        _fused_body,
        grid=(grid,),
        in_specs=[
            pl.BlockSpec((DIM, BLOCK), lambda i: (0, i)),
            pl.BlockSpec((BLOCK // 2, 2 * DIM), lambda i: (i, 0)),
            pl.BlockSpec((DIM, 2 * DIM), lambda i: (0, 0)),
            pl.BlockSpec((DIM, 1), lambda i: (0, 0)),
        ],
        out_specs=pl.BlockSpec((DIM, BLOCK), lambda i: (0, i)),
        out_shape=jax.ShapeDtypeStruct((DIM, batch), jnp.float32),
    )(xt, g128, W, b_col)


def kernel(x, t, time_embed, W, b):
    g128 = _sc_gather_packed(time_embed, t.astype(jnp.int32))
    out_t = _fused_matmul_t(
        jnp.transpose(x), g128, W, b.reshape(DIM, 1)
    )
    return jnp.transpose(out_t)


# speedup vs baseline: 2.0816x; 1.0759x over previous
import functools

import jax
import jax.numpy as jnp
from jax import lax
from jax.experimental import pallas as pl
from jax.experimental.pallas import tpu as pltpu
from jax.experimental.pallas import tpu_sc as plsc

DIM = 64
NUM_WORKERS = 32
GATHER_CHUNK = 128
BLOCK = 4096


def _sc_gather_packed(table, t):
    batch = t.shape[0]
    b_per_w = batch // NUM_WORKERS
    nchunks = b_per_w // GATHER_CHUNK
    mesh = plsc.VectorSubcoreMesh(core_axis_name="c", subcore_axis_name="s")

    @functools.partial(
        pl.kernel,
        mesh=mesh,
        compiler_params=pltpu.CompilerParams(use_tc_tiling_on_sc=False),
        out_type=jax.ShapeDtypeStruct((batch // 2, 2 * DIM), jnp.float32),
        scratch_types=[
            pltpu.VMEM((b_per_w,), jnp.int32),
            pltpu.VMEM((b_per_w, DIM), jnp.float32),
            pltpu.SemaphoreType.DMA,
        ],
    )
    def gather_kernel(tab_hbm, t_hbm, out_hbm, idx_v, rows_v, sem_g):
        wid = lax.axis_index("s") * 2 + lax.axis_index("c")
        base = wid * b_per_w
        rowbase = 1024 * (wid // 4) + 512 * (wid % 2)
        colbase = DIM * ((wid % 4) // 2)
        pltpu.sync_copy(t_hbm.at[pl.ds(base, b_per_w)], idx_v)
        gathers = [
            pltpu.async_copy(
                tab_hbm.at[idx_v.at[pl.ds(j * GATHER_CHUNK, GATHER_CHUNK)]],
                rows_v.at[pl.ds(j * GATHER_CHUNK, GATHER_CHUNK)],
                sem_g,
            )
            for j in range(nchunks)
        ]
        for g in gathers:
            g.wait()
        pltpu.sync_copy(
            rows_v,
            out_hbm.at[pl.ds(rowbase, b_per_w), pl.ds(colbase, DIM)],
        )

    return gather_kernel(table, t)


def _fused_body(xt_ref, g_ref, w_ref, b_ref, o_ref):
    xw = lax.dot_general(
        w_ref[:, :DIM], xt_ref[...],
        (((1,), (0,)), ((), ())),
        preferred_element_type=jnp.float32,
    )
    bias = b_ref[...]
    for sub in range(BLOCK // 2048):
        yev = lax.dot_general(
            w_ref[:, DIM:], g_ref[sub * 1024:(sub + 1) * 1024, :DIM],
            (((1,), (1,)), ((), ())),
            preferred_element_type=jnp.float32,
        )
        yod = lax.dot_general(
            w_ref[:, DIM:], g_ref[sub * 1024:(sub + 1) * 1024, DIM:],
            (((1,), (1,)), ((), ())),
            preferred_element_type=jnp.float32,
        )
        c0 = sub * 2048
        o_ref[:, c0:c0 + 1024] = xw[:, c0:c0 + 1024] + yev + bias
        o_ref[:, c0 + 1024:c0 + 2048] = xw[:, c0 + 1024:c0 + 2048] + yod + bias


def _fused_matmul_t(xt, g128, W, b_col):
    batch = xt.shape[1]
    grid = batch // BLOCK
    return pl.pallas_call(
        _fused_body,
        grid=(grid,),
        in_specs=[
            pl.BlockSpec((DIM, BLOCK), lambda i: (0, i)),
            pl.BlockSpec((BLOCK // 2, 2 * DIM), lambda i: (i, 0)),
            pl.BlockSpec((DIM, 2 * DIM), lambda i: (0, 0)),
            pl.BlockSpec((DIM, 1), lambda i: (0, 0)),
        ],
        out_specs=pl.BlockSpec((DIM, BLOCK), lambda i: (0, i)),
        out_shape=jax.ShapeDtypeStruct((DIM, batch), jnp.float32),
    )(xt, g128, W, b_col)


def kernel(x, t, time_embed, W, b):
    g128 = _sc_gather_packed(time_embed, t.astype(jnp.int32))
    out_t = _fused_matmul_t(
        jnp.transpose(x), g128, W, b.reshape(DIM, 1)
    )
    return jnp.transpose(out_t)


# speedup vs baseline: 2.0948x; 1.0064x over previous
import functools

import jax
import jax.numpy as jnp
from jax import lax
from jax.experimental import pallas as pl
from jax.experimental.pallas import tpu as pltpu
from jax.experimental.pallas import tpu_sc as plsc

DIM = 64
NUM_WORKERS = 32
GATHER_CHUNK = 128
BLOCK = 4096


def _sc_gather_packed(table, t):
    batch = t.shape[0]
    b_per_w = batch // NUM_WORKERS
    nchunks = b_per_w // GATHER_CHUNK
    mesh = plsc.VectorSubcoreMesh(core_axis_name="c", subcore_axis_name="s")

    @functools.partial(
        pl.kernel,
        mesh=mesh,
        compiler_params=pltpu.CompilerParams(
            use_tc_tiling_on_sc=False, skip_device_barrier=True
        ),
        out_type=jax.ShapeDtypeStruct((batch // 2, 2 * DIM), jnp.float32),
        scratch_types=[
            pltpu.VMEM((b_per_w,), jnp.int32),
            pltpu.VMEM((b_per_w, DIM), jnp.float32),
            pltpu.SemaphoreType.DMA,
        ],
    )
    def gather_kernel(tab_hbm, t_hbm, out_hbm, idx_v, rows_v, sem_g):
        wid = lax.axis_index("s") * 2 + lax.axis_index("c")
        base = wid * b_per_w
        rowbase = 1024 * (wid // 4) + 512 * (wid % 2)
        colbase = DIM * ((wid % 4) // 2)
        pltpu.sync_copy(t_hbm.at[pl.ds(base, b_per_w)], idx_v)
        gathers = [
            pltpu.async_copy(
                tab_hbm.at[idx_v.at[pl.ds(j * GATHER_CHUNK, GATHER_CHUNK)]],
                rows_v.at[pl.ds(j * GATHER_CHUNK, GATHER_CHUNK)],
                sem_g,
            )
            for j in range(nchunks)
        ]
        for g in gathers:
            g.wait()
        pltpu.sync_copy(
            rows_v,
            out_hbm.at[pl.ds(rowbase, b_per_w), pl.ds(colbase, DIM)],
        )

    return gather_kernel(table, t)


def _fused_body(xt_ref, g_ref, w_ref, b_ref, o_ref):
    xw = lax.dot_general(
        w_ref[:, :DIM], xt_ref[...],
        (((1,), (0,)), ((), ())),
        preferred_element_type=jnp.float32,
    )
    bias = b_ref[...]
    for sub in range(BLOCK // 2048):
        yev = lax.dot_general(
            w_ref[:, DIM:], g_ref[sub * 1024:(sub + 1) * 1024, :DIM],
            (((1,), (1,)), ((), ())),
            preferred_element_type=jnp.float32,
        )
        yod = lax.dot_general(
            w_ref[:, DIM:], g_ref[sub * 1024:(sub + 1) * 1024, DIM:],
            (((1,), (1,)), ((), ())),
            preferred_element_type=jnp.float32,
        )
        c0 = sub * 2048
        o_ref[:, c0:c0 + 1024] = xw[:, c0:c0 + 1024] + yev + bias
        o_ref[:, c0 + 1024:c0 + 2048] = xw[:, c0 + 1024:c0 + 2048] + yod + bias


def _fused_matmul_t(xt, g128, W, b_col):
    batch = xt.shape[1]
    grid = batch // BLOCK
    return pl.pallas_call(
        _fused_body,
        grid=(grid,),
        in_specs=[
            pl.BlockSpec((DIM, BLOCK), lambda i: (0, i)),
            pl.BlockSpec((BLOCK // 2, 2 * DIM), lambda i: (i, 0)),
            pl.BlockSpec((DIM, 2 * DIM), lambda i: (0, 0)),
            pl.BlockSpec((DIM, 1), lambda i: (0, 0)),
        ],
        out_specs=pl.BlockSpec((DIM, BLOCK), lambda i: (0, i)),
        out_shape=jax.ShapeDtypeStruct((DIM, batch), jnp.float32),
    )(xt, g128, W, b_col)


def kernel(x, t, time_embed, W, b):
    g128 = _sc_gather_packed(time_embed, t.astype(jnp.int32))
    out_t = _fused_matmul_t(
        jnp.transpose(x), g128, W, b.reshape(DIM, 1)
    )
    return jnp.transpose(out_t)
